# baseline (speedup 1.0000x reference)
import functools
import math

import jax
import jax.numpy as jnp
from jax import lax
from jax.experimental import pallas as pl
from jax.experimental.pallas import tpu as pltpu
from jax.experimental.pallas import tpu_sc as plsc

VOCAB = 1000000
DIM = 32
SEQ = 200
NB = 4096
HALF = 16

NC = 2
NS = 16
DPC = DIM // NC
BS = 128
SLICES = NB // (NS * BS)
BLK = 50
NBLK = SEQ // BLK

_SCALE = math.sqrt(float(DIM))


def _pe_table(length, depth):
    half = depth // 2
    positions = jnp.arange(length, dtype=jnp.float32).reshape(-1, 1)
    depths = jnp.arange(half, dtype=jnp.float32).reshape(1, -1) / half
    angle = positions / (10000.0 ** depths)
    return jnp.concatenate([jnp.sin(angle), jnp.cos(angle)], axis=-1)


@functools.lru_cache(maxsize=None)
def _make_kernel():
    mesh = plsc.VectorSubcoreMesh(core_axis_name="c", subcore_axis_name="s")

    @functools.partial(
        pl.kernel,
        mesh=mesh,
        out_type=jax.ShapeDtypeStruct((SEQ, DIM, NB), jnp.float32),
        scratch_types=[
            pltpu.VMEM((SLICES, SEQ, BS), jnp.int32),
            pltpu.VMEM((BLK, BS), jnp.float32),
            pltpu.VMEM((BLK, BS), jnp.float32),
            pltpu.VMEM((SEQ, HALF), jnp.float32),
            pltpu.VMEM_SHARED((VOCAB,), jnp.float32),
            pltpu.SemaphoreType.DMA,
            pltpu.SemaphoreType.DMA,
            pltpu.SemaphoreType.DMA,
        ],
        compiler_params=pltpu.CompilerParams(use_tc_tiling_on_sc=False),
    )
    def emb(xt_hbm, tt_hbm, pe_hbm, out_hbm, idx_v, val_a, val_b, pe_v,
            row_sh, gsem, osem_a, osem_b):
        cid = lax.axis_index("c")
        sid = lax.axis_index("s")
        scale = jnp.float32(_SCALE)
        vals = (val_a, val_b)
        osems = (osem_a, osem_b)

        for h in range(SLICES):
            b0 = (sid * SLICES + h) * BS
            pltpu.sync_copy(xt_hbm.at[:, pl.ds(b0, BS)], idx_v.at[h])

        def unit(u, carry):
            dd = u // SLICES
            h = u % SLICES
            d = cid * DPC + dd
            b0 = (sid * SLICES + h) * BS

            @pl.when(h == 0)
            def _swap_row():
                plsc.subcore_barrier()

                @pl.when(sid == 0)
                def _load():
                    pltpu.sync_copy(tt_hbm.at[d], row_sh)

                pltpu.sync_copy(pe_hbm.at[d], pe_v)
                plsc.subcore_barrier()

            def out_blk(j):
                return out_hbm.at[pl.ds(j * BLK, BLK), d, pl.ds(b0, BS)]

            def fire(j):
                dst = vals[j % 2]
                for k in range(BLK):
                    pltpu.async_copy(
                        row_sh.at[idx_v.at[h, j * BLK + k]], dst.at[k], gsem
                    )

            def drain_gather(j):
                pltpu.make_async_copy(out_blk(j), vals[j % 2], gsem).wait()

            def drain_out(j, guarded):
                cp = pltpu.make_async_copy(out_blk(j), vals[j % 2],
                                           osems[j % 2])
                if guarded:
                    @pl.when(u > 0)
                    def _w():
                        cp.wait()
                else:
                    cp.wait()

            def compute(j):
                dst = vals[j % 2]

                def body(sl, c2):
                    pe_vec = pe_v[j * BLK + sl, pl.ds(0, HALF)]
                    for jj in range(BS // HALF):
                        dst[sl, pl.ds(jj * HALF, HALF)] = (
                            dst[sl, pl.ds(jj * HALF, HALF)] * scale + pe_vec
                        )
                    return c2

                lax.fori_loop(0, BLK, body, 0, unroll=False)

            def store(j):
                pltpu.async_copy(vals[j % 2], out_blk(j), osems[j % 2])

            drain_out(0, guarded=True)
            fire(0)
            drain_gather(0)
            drain_out(1, guarded=True)
            fire(1)
            compute(0)
            store(0)
            for j in range(1, NBLK):
                drain_gather(j)
                if j < NBLK - 1:
                    drain_out(j + 1, guarded=False)
                    fire(j + 1)
                compute(j)
                store(j)
            return carry

        lax.fori_loop(0, DPC * SLICES, unit, 0, unroll=False)
        pltpu.make_async_copy(
            out_hbm.at[pl.ds(0, BLK), 0, pl.ds(0, BS)], val_a, osem_a
        ).wait()
        pltpu.make_async_copy(
            out_hbm.at[pl.ds(0, BLK), 0, pl.ds(0, BS)], val_b, osem_b
        ).wait()

    return emb


def kernel(x, table):
    batch, seq = x.shape
    vocab, dim = table.shape
    assert (batch, seq, vocab, dim) == (NB, SEQ, VOCAB, DIM)
    pe = _pe_table(seq, dim)
    pe_b = jnp.broadcast_to(pe.T[:, :, None], (dim, seq, HALF))
    out = _make_kernel()(x.T, table.T, pe_b)
    return jnp.transpose(out, (2, 0, 1))


# speedup vs baseline: 7.5833x; 7.5833x over previous
import functools
import math

import jax
import jax.numpy as jnp
from jax import lax
from jax.experimental import pallas as pl
from jax.experimental.pallas import tpu as pltpu
from jax.experimental.pallas import tpu_sc as plsc

VOCAB = 1000000
DIM = 32
SEQ = 200
NB = 4096
HALF = 16

NC = 2
NS = 16
DPC = DIM // NC
BS = 128
SLICES = NB // (NS * BS)
BLK = 40
NBLK = SEQ // BLK

_SCALE = math.sqrt(float(DIM))


def _pe_table(length, depth):
    half = depth // 2
    positions = jnp.arange(length, dtype=jnp.float32).reshape(-1, 1)
    depths = jnp.arange(half, dtype=jnp.float32).reshape(1, -1) / half
    angle = positions / (10000.0 ** depths)
    return jnp.concatenate([jnp.sin(angle), jnp.cos(angle)], axis=-1)


@functools.lru_cache(maxsize=None)
def _make_kernel():
    mesh = plsc.VectorSubcoreMesh(core_axis_name="c", subcore_axis_name="s")

    @functools.partial(
        pl.kernel,
        mesh=mesh,
        out_type=jax.ShapeDtypeStruct((SEQ, DIM, NB), jnp.float32),
        scratch_types=[
            pltpu.VMEM((SLICES, SEQ, BS), jnp.int32),
            pltpu.VMEM((BLK, BS), jnp.float32),
            pltpu.VMEM((BLK, BS), jnp.float32),
            pltpu.VMEM((SEQ // 8, 128), jnp.float32),
            pltpu.VMEM_SHARED((VOCAB,), jnp.float32),
            pltpu.SemaphoreType.DMA,
            pltpu.SemaphoreType.DMA,
            pltpu.SemaphoreType.DMA,
        ],
    )
    def emb(xt_hbm, tt_hbm, pe_hbm, out_hbm, idx_v, val_a, val_b, pe_v,
            row_sh, gsem, osem_a, osem_b):
        cid = lax.axis_index("c")
        sid = lax.axis_index("s")
        scale = jnp.float32(_SCALE)
        vals = (val_a, val_b)
        osems = (osem_a, osem_b)

        for h in range(SLICES):
            b0 = (sid * SLICES + h) * BS
            pltpu.sync_copy(xt_hbm.at[:, pl.ds(b0, BS)], idx_v.at[h])

        def unit(u, carry):
            dd = u // SLICES
            h = u % SLICES
            d = cid * DPC + dd
            b0 = (sid * SLICES + h) * BS

            @pl.when(h == 0)
            def _swap_row():
                plsc.subcore_barrier()

                @pl.when(sid == 0)
                def _load():
                    pltpu.sync_copy(tt_hbm.at[d], row_sh)

                pltpu.sync_copy(pe_hbm.at[d], pe_v)
                plsc.subcore_barrier()

            def out_blk(j):
                return out_hbm.at[pl.ds(j * BLK, BLK), d, pl.ds(b0, BS)]

            def fire(j):
                dst = vals[j % 2]
                for k in range(BLK):
                    pltpu.async_copy(
                        row_sh.at[idx_v.at[h, j * BLK + k]], dst.at[k], gsem
                    )

            def drain_gather(j):
                pltpu.make_async_copy(out_blk(j), vals[j % 2], gsem).wait()

            def drain_out(j, guarded):
                cp = pltpu.make_async_copy(out_blk(j), vals[j % 2],
                                           osems[j % 2])
                if guarded:
                    @pl.when(u > 0)
                    def _w():
                        cp.wait()
                else:
                    cp.wait()

            def compute(j):
                dst = vals[j % 2]

                def body(sl, c2):
                    s_glob = j * BLK + sl
                    pe_vec = pe_v[s_glob // 8, pl.ds((s_glob % 8) * HALF, HALF)]
                    for jj in range(BS // HALF):
                        dst[sl, pl.ds(jj * HALF, HALF)] = (
                            dst[sl, pl.ds(jj * HALF, HALF)] * scale + pe_vec
                        )
                    return c2

                lax.fori_loop(0, BLK, body, 0, unroll=False)

            def store(j):
                pltpu.async_copy(vals[j % 2], out_blk(j), osems[j % 2])

            drain_out(0, guarded=True)
            fire(0)
            drain_gather(0)
            drain_out(1, guarded=True)
            fire(1)
            compute(0)
            store(0)
            for j in range(1, NBLK):
                drain_gather(j)
                if j < NBLK - 1:
                    drain_out(j + 1, guarded=False)
                    fire(j + 1)
                compute(j)
                store(j)
            return carry

        lax.fori_loop(0, DPC * SLICES, unit, 0, unroll=False)
        pltpu.make_async_copy(
            out_hbm.at[pl.ds(0, BLK), 0, pl.ds(0, BS)], val_a, osem_a
        ).wait()
        pltpu.make_async_copy(
            out_hbm.at[pl.ds(0, BLK), 0, pl.ds(0, BS)], val_b, osem_b
        ).wait()

    return emb


def kernel(x, table):
    batch, seq = x.shape
    vocab, dim = table.shape
    assert (batch, seq, vocab, dim) == (NB, SEQ, VOCAB, DIM)
    pe = _pe_table(seq, dim)
    pe_b = jnp.broadcast_to(
        pe.T.reshape(dim, seq // 8, 8, 1), (dim, seq // 8, 8, HALF)
    ).reshape(dim, seq // 8, 8 * HALF)
    out = _make_kernel()(x.T, table.T, pe_b)
    return jnp.transpose(out, (2, 0, 1))
